# baseline (speedup 1.0000x reference)
import functools

import jax
import jax.numpy as jnp
from jax import lax
from jax.experimental import pallas as pl
from jax.experimental.pallas import tpu as pltpu
from jax.experimental.pallas import tpu_sc as plsc

B, L, H = 4, 2048, 4096
N_EVID = 128
TOP_K = 3

_CL = 512
_CH = 512
_NSTEP = H // _CH

_NC, _NS = 2, 16
_NW = _NC * _NS
_ROWS = B * N_EVID
_RPW = _ROWS // _NW


def _sc_gather_evidence(table, idx):
    mesh = plsc.VectorSubcoreMesh(core_axis_name="c", subcore_axis_name="s")

    @functools.partial(
        pl.kernel,
        mesh=mesh,
        out_type=jax.ShapeDtypeStruct((_ROWS, H), jnp.float32),
        scratch_types=[
            pltpu.VMEM((_RPW,), jnp.int32),
            pltpu.VMEM((_RPW, H), jnp.float32),
            pltpu.SemaphoreType.DMA,
        ],
    )
    def k(table_hbm, idx_hbm, out_hbm, idx_v, rows_v, sem):
        wid = lax.axis_index("s") * _NC + lax.axis_index("c")
        base = wid * _RPW
        pltpu.sync_copy(idx_hbm.at[pl.ds(base, _RPW)], idx_v)
        pltpu.async_copy(table_hbm.at[idx_v], rows_v, sem).wait()
        pltpu.sync_copy(rows_v, out_hbm.at[pl.ds(base, _RPW)])

    return k(table, idx)


def _sc_gather_topk(evidence, idx16):
    mesh = plsc.VectorSubcoreMesh(core_axis_name="c", subcore_axis_name="s")

    @functools.partial(
        pl.kernel,
        mesh=mesh,
        out_type=jax.ShapeDtypeStruct((16, H), jnp.float32),
        scratch_types=[
            pltpu.VMEM((16,), jnp.int32),
            pltpu.VMEM((16, H), jnp.float32),
            pltpu.SemaphoreType.DMA,
        ],
    )
    def k(ev_hbm, idx_hbm, out_hbm, idx_v, rows_v, sem):
        wid = lax.axis_index("s") * _NC + lax.axis_index("c")

        @pl.when(wid == 0)
        def _():
            pltpu.sync_copy(idx_hbm, idx_v)
            pltpu.async_copy(ev_hbm.at[idx_v], rows_v, sem).wait()
            pltpu.sync_copy(rows_v, out_hbm)

    return k(evidence, idx16)


def _pool_body(qe_ref, txt_ref, img_ref, pooled_ref, wsum_ref, isum_ref,
               tsum_ref):
    j = pl.program_id(1)

    @pl.when(j == 0)
    def _():
        pooled_ref[...] = jnp.zeros_like(pooled_ref)
        wsum_ref[...] = jnp.zeros_like(wsum_ref)
        isum_ref[...] = jnp.zeros_like(isum_ref)
        tsum_ref[...] = jnp.zeros_like(tsum_ref)

    t = txt_ref[0, :, pl.ds(j * _CL, _CL)]
    im = img_ref[0, :, pl.ds(j * _CL, _CL)]
    w = 1.0 - t
    qe = qe_ref[0]
    pooled_ref[0] += jnp.dot(w, qe, preferred_element_type=jnp.float32)
    wsum_ref[0] += jnp.broadcast_to(
        jnp.sum(w, axis=1, keepdims=True), (1, 128))
    isum_ref[0] += jnp.broadcast_to(
        jnp.sum(im, axis=1, keepdims=True), (1, 128))
    tsum_ref[0] += jnp.broadcast_to(
        jnp.sum(t, axis=1, keepdims=True), (1, 128))


def _pool(qe, txt3, img3):
    return pl.pallas_call(
        _pool_body,
        grid=(B, L // _CL),
        in_specs=[
            pl.BlockSpec((1, _CL, H), lambda b, j: (b, j, 0)),
            pl.BlockSpec((1, 1, L), lambda b, j: (b, 0, 0)),
            pl.BlockSpec((1, 1, L), lambda b, j: (b, 0, 0)),
        ],
        out_specs=[
            pl.BlockSpec((1, 1, H), lambda b, j: (b, 0, 0)),
            pl.BlockSpec((1, 1, 128), lambda b, j: (b, 0, 0)),
            pl.BlockSpec((1, 1, 128), lambda b, j: (b, 0, 0)),
            pl.BlockSpec((1, 1, 128), lambda b, j: (b, 0, 0)),
        ],
        out_shape=[
            jax.ShapeDtypeStruct((B, 1, H), jnp.float32),
            jax.ShapeDtypeStruct((B, 1, 128), jnp.float32),
            jax.ShapeDtypeStruct((B, 1, 128), jnp.float32),
            jax.ShapeDtypeStruct((B, 1, 128), jnp.float32),
        ],
        compiler_params=pltpu.CompilerParams(
            dimension_semantics=("arbitrary", "arbitrary")),
    )(qe, txt3, img3)


def _score_body(ev_ref, we_ref, wq_ref, pooled_ref, wsum_ref, isum_ref,
                tsum_ref, bq_ref, be_ref, scores_ref, idx_ref,
                proj_acc, qacc):
    j = pl.program_id(0)

    @pl.when(j == 0)
    def _():
        proj_acc[...] = jnp.zeros_like(proj_acc)
        qacc[...] = jnp.zeros_like(qacc)

    nt = (((1,), (1,)), ((), ()))
    proj_acc[...] += lax.dot_general(
        ev_ref[...], we_ref[...], nt, preferred_element_type=jnp.float32)
    qacc[...] += lax.dot_general(
        pooled_ref[...], wq_ref[...], nt, preferred_element_type=jnp.float32)

    @pl.when(j == _NSTEP - 1)
    def _():
        eps = 1e-8
        wcol = wsum_ref[...][:, 0:1]
        icol = isum_ref[...][:, 0:1]
        tcol = tsum_ref[...][:, 0:1]
        inv_w = 1.0 / (wcol + 1e-6)
        query = qacc[...] * inv_w + bq_ref[...]
        qn = jnp.maximum(
            jnp.sqrt(jnp.sum(query * query, axis=1, keepdims=True)), eps)
        noise = 2.0 - icol * (1.0 / L)
        att = 0.5 + 0.5 * tcol * (1.0 / L)
        scale = noise * att / qn

        proj = proj_acc[...] + be_ref[...]
        p3 = proj.reshape(B, N_EVID, H)
        pn2 = jnp.sum(p3 * p3, axis=2)
        qd = jnp.sum(p3 * query.reshape(B, 1, H), axis=2)
        pn = jnp.maximum(jnp.sqrt(pn2), eps)
        scores = qd / pn * scale

        lane = lax.broadcasted_iota(jnp.int32, (B, N_EVID), 1)
        cur = scores
        out_s = jnp.zeros((B, N_EVID), jnp.float32)
        out_i = jnp.zeros((B, N_EVID), jnp.int32)
        for k in range(TOP_K):
            m = jnp.max(cur, axis=1, keepdims=True)
            am = jnp.min(jnp.where(cur == m, lane, N_EVID), axis=1,
                         keepdims=True)
            out_s = jnp.where(lane == k, m, out_s)
            out_i = jnp.where(lane == k, am, out_i)
            cur = jnp.where(lane == am, -jnp.inf, cur)
        scores_ref[...] = out_s
        idx_ref[...] = out_i


def _score(evidence, We, Wq, pooled, wsum, isum, tsum, bq2, be2):
    return pl.pallas_call(
        _score_body,
        grid=(_NSTEP,),
        in_specs=[
            pl.BlockSpec((_ROWS, _CH), lambda j: (0, j)),
            pl.BlockSpec((H, _CH), lambda j: (0, j)),
            pl.BlockSpec((H, _CH), lambda j: (0, j)),
            pl.BlockSpec((B, _CH), lambda j: (0, j)),
            pl.BlockSpec((B, 128), lambda j: (0, 0)),
            pl.BlockSpec((B, 128), lambda j: (0, 0)),
            pl.BlockSpec((B, 128), lambda j: (0, 0)),
            pl.BlockSpec((1, H), lambda j: (0, 0)),
            pl.BlockSpec((1, H), lambda j: (0, 0)),
        ],
        out_specs=[
            pl.BlockSpec((B, N_EVID), lambda j: (0, 0)),
            pl.BlockSpec((B, N_EVID), lambda j: (0, 0)),
        ],
        out_shape=[
            jax.ShapeDtypeStruct((B, N_EVID), jnp.float32),
            jax.ShapeDtypeStruct((B, N_EVID), jnp.int32),
        ],
        scratch_shapes=[
            pltpu.VMEM((_ROWS, H), jnp.float32),
            pltpu.VMEM((B, H), jnp.float32),
        ],
        compiler_params=pltpu.CompilerParams(
            dimension_semantics=("arbitrary",)),
    )(evidence, We, Wq, pooled, wsum, isum, tsum, bq2, be2)


def kernel(question_embeddings, evidence_tokens, img_conf, txt_conf,
           emb_table, Wq, bq, We, be):
    tokens = evidence_tokens.reshape(-1).astype(jnp.int32)
    evidence = _sc_gather_evidence(emb_table, tokens)

    txt3 = txt_conf.reshape(B, 1, L)
    img3 = img_conf.reshape(B, 1, L)
    pooled, wsum, isum, tsum = _pool(question_embeddings, txt3, img3)
    pooled = pooled.reshape(B, H)
    wsum = wsum.reshape(B, 128)
    isum = isum.reshape(B, 128)
    tsum = tsum.reshape(B, 128)

    scores128, idx128 = _score(evidence, We, Wq, pooled, wsum, isum, tsum,
                               bq.reshape(1, H), be.reshape(1, H))
    topk_scores = scores128[:, :TOP_K]
    topk_idx = idx128[:, :TOP_K]

    flat = (jnp.arange(B, dtype=jnp.int32)[:, None] * N_EVID
            + topk_idx).reshape(-1)
    idx16 = jnp.concatenate([flat, jnp.zeros((4,), jnp.int32)])
    rows16 = _sc_gather_topk(evidence, idx16)
    topk_emb = rows16[: B * TOP_K].reshape(B, TOP_K, 1, H)
    return (topk_emb, topk_scores)


# speedup vs baseline: 1.0569x; 1.0569x over previous
import functools

import jax
import jax.numpy as jnp
from jax import lax
from jax.experimental import pallas as pl
from jax.experimental.pallas import tpu as pltpu
from jax.experimental.pallas import tpu_sc as plsc

B, L, H = 4, 2048, 4096
N_EVID = 128
TOP_K = 3

_CL = 512
_CH = 512
_NSTEP = H // _CH

_NC, _NS = 2, 16
_NW = _NC * _NS
_ROWS = B * N_EVID
_RPW = _ROWS // _NW


def _sc_gather_evidence(table, idx):
    mesh = plsc.VectorSubcoreMesh(core_axis_name="c", subcore_axis_name="s")

    @functools.partial(
        pl.kernel,
        mesh=mesh,
        out_type=jax.ShapeDtypeStruct((_ROWS, H), jnp.float32),
        scratch_types=[
            pltpu.VMEM((_RPW,), jnp.int32),
            pltpu.VMEM((_RPW, H), jnp.float32),
            pltpu.SemaphoreType.DMA,
        ],
    )
    def k(table_hbm, idx_hbm, out_hbm, idx_v, rows_v, sem):
        wid = lax.axis_index("s") * _NC + lax.axis_index("c")
        base = wid * _RPW
        pltpu.sync_copy(idx_hbm.at[pl.ds(base, _RPW)], idx_v)
        pltpu.async_copy(table_hbm.at[idx_v], rows_v, sem).wait()
        pltpu.sync_copy(rows_v, out_hbm.at[pl.ds(base, _RPW)])

    return k(table, idx)


def _pool_body(qe_ref, txt_ref, img_ref, pooled_ref, wsum_ref, isum_ref,
               tsum_ref):
    j = pl.program_id(1)

    @pl.when(j == 0)
    def _():
        pooled_ref[...] = jnp.zeros_like(pooled_ref)
        wsum_ref[...] = jnp.zeros_like(wsum_ref)
        isum_ref[...] = jnp.zeros_like(isum_ref)
        tsum_ref[...] = jnp.zeros_like(tsum_ref)

    t = txt_ref[0, :, pl.ds(j * _CL, _CL)]
    im = img_ref[0, :, pl.ds(j * _CL, _CL)]
    w = 1.0 - t
    qe = qe_ref[0]
    pooled_ref[0] += jnp.dot(w, qe, preferred_element_type=jnp.float32)
    wsum_ref[0] += jnp.broadcast_to(
        jnp.sum(w, axis=1, keepdims=True), (1, 128))
    isum_ref[0] += jnp.broadcast_to(
        jnp.sum(im, axis=1, keepdims=True), (1, 128))
    tsum_ref[0] += jnp.broadcast_to(
        jnp.sum(t, axis=1, keepdims=True), (1, 128))


def _pool(qe, txt3, img3):
    return pl.pallas_call(
        _pool_body,
        grid=(B, L // _CL),
        in_specs=[
            pl.BlockSpec((1, _CL, H), lambda b, j: (b, j, 0)),
            pl.BlockSpec((1, 1, L), lambda b, j: (b, 0, 0)),
            pl.BlockSpec((1, 1, L), lambda b, j: (b, 0, 0)),
        ],
        out_specs=[
            pl.BlockSpec((1, 1, H), lambda b, j: (b, 0, 0)),
            pl.BlockSpec((1, 1, 128), lambda b, j: (b, 0, 0)),
            pl.BlockSpec((1, 1, 128), lambda b, j: (b, 0, 0)),
            pl.BlockSpec((1, 1, 128), lambda b, j: (b, 0, 0)),
        ],
        out_shape=[
            jax.ShapeDtypeStruct((B, 1, H), jnp.float32),
            jax.ShapeDtypeStruct((B, 1, 128), jnp.float32),
            jax.ShapeDtypeStruct((B, 1, 128), jnp.float32),
            jax.ShapeDtypeStruct((B, 1, 128), jnp.float32),
        ],
        compiler_params=pltpu.CompilerParams(
            dimension_semantics=("arbitrary", "arbitrary")),
    )(qe, txt3, img3)


def _score_body(ev_ref, we_ref, wq_ref, pooled_ref, wsum_ref, isum_ref,
                tsum_ref, bq_ref, be_ref, scores_ref, emb_ref,
                proj_acc, qacc):
    j = pl.program_id(0)

    @pl.when(j == 0)
    def _():
        proj_acc[...] = jnp.zeros_like(proj_acc)
        qacc[...] = jnp.zeros_like(qacc)

    nt = (((1,), (1,)), ((), ()))
    ev_blk = ev_ref[:, pl.ds(j * _CH, _CH)]
    proj_acc[...] += lax.dot_general(
        ev_blk, we_ref[...], nt, preferred_element_type=jnp.float32)
    qacc[...] += lax.dot_general(
        pooled_ref[...], wq_ref[...], nt, preferred_element_type=jnp.float32)

    @pl.when(j == _NSTEP - 1)
    def _():
        eps = 1e-8
        wcol = wsum_ref[...][:, 0:1]
        icol = isum_ref[...][:, 0:1]
        tcol = tsum_ref[...][:, 0:1]
        inv_w = 1.0 / (wcol + 1e-6)
        query = qacc[...] * inv_w + bq_ref[...]
        qn = jnp.maximum(
            jnp.sqrt(jnp.sum(query * query, axis=1, keepdims=True)), eps)
        noise = 2.0 - icol * (1.0 / L)
        att = 0.5 + 0.5 * tcol * (1.0 / L)
        scale = noise * att / qn

        proj = proj_acc[...] + be_ref[...]
        p3 = proj.reshape(B, N_EVID, H)
        pn2 = jnp.sum(p3 * p3, axis=2)
        qd = jnp.sum(p3 * query.reshape(B, 1, H), axis=2)
        pn = jnp.maximum(jnp.sqrt(pn2), eps)
        scores = qd / pn * scale

        lane = lax.broadcasted_iota(jnp.int32, (1, N_EVID), 1)
        for b in range(B):
            cur = scores[b:b + 1, :]
            out_row = jnp.zeros((1, N_EVID), jnp.float32)
            for k in range(TOP_K):
                m = jnp.max(cur)
                am = jnp.min(jnp.where(cur == m, lane, N_EVID))
                out_row = jnp.where(lane == k, m, out_row)
                cur = jnp.where(lane == am, -jnp.inf, cur)
                emb_ref[pl.ds(b * TOP_K + k, 1), :] = (
                    ev_ref[pl.ds(b * N_EVID + am, 1), :])
            scores_ref[pl.ds(b, 1), :] = out_row


def _score(evidence, We, Wq, pooled, wsum, isum, tsum, bq2, be2):
    return pl.pallas_call(
        _score_body,
        grid=(_NSTEP,),
        in_specs=[
            pl.BlockSpec((_ROWS, H), lambda j: (0, 0)),
            pl.BlockSpec((H, _CH), lambda j: (0, j)),
            pl.BlockSpec((H, _CH), lambda j: (0, j)),
            pl.BlockSpec((B, _CH), lambda j: (0, j)),
            pl.BlockSpec((B, 128), lambda j: (0, 0)),
            pl.BlockSpec((B, 128), lambda j: (0, 0)),
            pl.BlockSpec((B, 128), lambda j: (0, 0)),
            pl.BlockSpec((1, H), lambda j: (0, 0)),
            pl.BlockSpec((1, H), lambda j: (0, 0)),
        ],
        out_specs=[
            pl.BlockSpec((B, N_EVID), lambda j: (0, 0)),
            pl.BlockSpec((B * TOP_K + 4, H), lambda j: (0, 0)),
        ],
        out_shape=[
            jax.ShapeDtypeStruct((B, N_EVID), jnp.float32),
            jax.ShapeDtypeStruct((B * TOP_K + 4, H), jnp.float32),
        ],
        scratch_shapes=[
            pltpu.VMEM((_ROWS, H), jnp.float32),
            pltpu.VMEM((B, H), jnp.float32),
        ],
        compiler_params=pltpu.CompilerParams(
            dimension_semantics=("arbitrary",)),
    )(evidence, We, Wq, pooled, wsum, isum, tsum, bq2, be2)


def kernel(question_embeddings, evidence_tokens, img_conf, txt_conf,
           emb_table, Wq, bq, We, be):
    tokens = evidence_tokens.reshape(-1).astype(jnp.int32)
    evidence = _sc_gather_evidence(emb_table, tokens)

    txt3 = txt_conf.reshape(B, 1, L)
    img3 = img_conf.reshape(B, 1, L)
    pooled, wsum, isum, tsum = _pool(question_embeddings, txt3, img3)
    pooled = pooled.reshape(B, H)
    wsum = wsum.reshape(B, 128)
    isum = isum.reshape(B, 128)
    tsum = tsum.reshape(B, 128)

    scores128, emb16 = _score(evidence, We, Wq, pooled, wsum, isum, tsum,
                              bq.reshape(1, H), be.reshape(1, H))
    topk_scores = scores128[:, :TOP_K]
    topk_emb = emb16[: B * TOP_K].reshape(B, TOP_K, 1, H)
    return (topk_emb, topk_scores)


# speedup vs baseline: 1.0750x; 1.0172x over previous
import functools

import jax
import jax.numpy as jnp
from jax import lax
from jax.experimental import pallas as pl
from jax.experimental.pallas import tpu as pltpu
from jax.experimental.pallas import tpu_sc as plsc

B, L, H = 4, 2048, 4096
N_EVID = 128
TOP_K = 3

_CH = 256
_NSTEP = H // _CH

_NC, _NS = 2, 16
_NW = _NC * _NS
_ROWS = B * N_EVID
_RPW = _ROWS // _NW


def _sc_gather_evidence(table, idx):
    mesh = plsc.VectorSubcoreMesh(core_axis_name="c", subcore_axis_name="s")

    @functools.partial(
        pl.kernel,
        mesh=mesh,
        out_type=jax.ShapeDtypeStruct((_ROWS, H), jnp.float32),
        scratch_types=[
            pltpu.VMEM((_RPW,), jnp.int32),
            pltpu.VMEM((_RPW, H), jnp.float32),
            pltpu.SemaphoreType.DMA,
        ],
    )
    def k(table_hbm, idx_hbm, out_hbm, idx_v, rows_v, sem):
        wid = lax.axis_index("s") * _NC + lax.axis_index("c")
        base = wid * _RPW
        pltpu.sync_copy(idx_hbm.at[pl.ds(base, _RPW)], idx_v)
        pltpu.async_copy(table_hbm.at[idx_v], rows_v, sem).wait()
        pltpu.sync_copy(rows_v, out_hbm.at[pl.ds(base, _RPW)])

    return k(table, idx)


def _fused_body(qe_ref, txt_ref, img_ref, ev_ref, we_ref, wq_ref, bq_ref,
                be_ref, scores_ref, emb_ref, proj_acc, qacc):
    c = pl.program_id(0)

    @pl.when(c == 0)
    def _():
        proj_acc[...] = jnp.zeros_like(proj_acc)
        qacc[...] = jnp.zeros_like(qacc)

    nt = (((1,), (1,)), ((), ()))
    t = txt_ref[:, 0, :]
    w = 1.0 - t
    qe3 = qe_ref[...]
    pooled_c = lax.dot_general(
        w.reshape(B, 1, L), qe3, (((2,), (1,)), ((0,), (0,))),
        preferred_element_type=jnp.float32).reshape(B, _CH)

    ev_blk = ev_ref[:, pl.ds(c * _CH, _CH)]
    proj_acc[...] += lax.dot_general(
        ev_blk, we_ref[...], nt, preferred_element_type=jnp.float32)
    qacc[...] += lax.dot_general(
        pooled_c, wq_ref[...], nt, preferred_element_type=jnp.float32)

    @pl.when(c == _NSTEP - 1)
    def _():
        eps = 1e-8
        im = img_ref[:, 0, :]
        wcol = jnp.sum(w, axis=1, keepdims=True)
        icol = jnp.sum(im, axis=1, keepdims=True)
        tcol = jnp.sum(t, axis=1, keepdims=True)
        inv_w = 1.0 / (wcol + 1e-6)
        query = qacc[...] * inv_w + bq_ref[...]
        qn = jnp.maximum(
            jnp.sqrt(jnp.sum(query * query, axis=1, keepdims=True)), eps)
        noise = 2.0 - icol * (1.0 / L)
        att = 0.5 + 0.5 * tcol * (1.0 / L)
        scale = noise * att / qn

        proj = proj_acc[...] + be_ref[...]
        p3 = proj.reshape(B, N_EVID, H)
        pn2 = jnp.sum(p3 * p3, axis=2)
        qd = jnp.sum(p3 * query.reshape(B, 1, H), axis=2)
        pn = jnp.maximum(jnp.sqrt(pn2), eps)
        scores = qd / pn * scale

        lane = lax.broadcasted_iota(jnp.int32, (1, N_EVID), 1)
        for b in range(B):
            cur = scores[b:b + 1, :]
            out_row = jnp.zeros((1, N_EVID), jnp.float32)
            for k in range(TOP_K):
                m = jnp.max(cur)
                am = jnp.min(jnp.where(cur == m, lane, N_EVID))
                out_row = jnp.where(lane == k, m, out_row)
                cur = jnp.where(lane == am, -jnp.inf, cur)
                emb_ref[b, k] = ev_ref[pl.ds(b * N_EVID + am, 1), :]
            scores_ref[pl.ds(b, 1), :] = out_row


def _fused(qe, txt3, img3, evidence, We, Wq, bq2, be2):
    return pl.pallas_call(
        _fused_body,
        grid=(_NSTEP,),
        in_specs=[
            pl.BlockSpec((B, L, _CH), lambda c: (0, 0, c)),
            pl.BlockSpec((B, 1, L), lambda c: (0, 0, 0)),
            pl.BlockSpec((B, 1, L), lambda c: (0, 0, 0)),
            pl.BlockSpec((_ROWS, H), lambda c: (0, 0)),
            pl.BlockSpec((H, _CH), lambda c: (0, c)),
            pl.BlockSpec((H, _CH), lambda c: (0, c)),
            pl.BlockSpec((1, H), lambda c: (0, 0)),
            pl.BlockSpec((1, H), lambda c: (0, 0)),
        ],
        out_specs=[
            pl.BlockSpec((B, N_EVID), lambda c: (0, 0)),
            pl.BlockSpec((B, TOP_K, 1, H), lambda c: (0, 0, 0, 0)),
        ],
        out_shape=[
            jax.ShapeDtypeStruct((B, N_EVID), jnp.float32),
            jax.ShapeDtypeStruct((B, TOP_K, 1, H), jnp.float32),
        ],
        scratch_shapes=[
            pltpu.VMEM((_ROWS, H), jnp.float32),
            pltpu.VMEM((B, H), jnp.float32),
        ],
        compiler_params=pltpu.CompilerParams(
            dimension_semantics=("arbitrary",)),
    )(qe, txt3, img3, evidence, We, Wq, bq2, be2)


def kernel(question_embeddings, evidence_tokens, img_conf, txt_conf,
           emb_table, Wq, bq, We, be):
    tokens = evidence_tokens.reshape(-1).astype(jnp.int32)
    evidence = _sc_gather_evidence(emb_table, tokens)

    txt3 = txt_conf.reshape(B, 1, L)
    img3 = img_conf.reshape(B, 1, L)
    scores128, topk_emb = _fused(question_embeddings, txt3, img3, evidence,
                                 We, Wq, bq.reshape(1, H), be.reshape(1, H))
    topk_scores = scores128[:, :TOP_K]
    return (topk_emb, topk_scores)
